# baseline (speedup 1.0000x reference)
import jax
import jax.numpy as jnp
from jax import lax
from jax.experimental import pallas as pl
from jax.experimental.pallas import tpu as pltpu
from jax.experimental.pallas import tpu_sc as plsc

_NUM_CLASSES = 100000
_FEAT_DIM = 64
_BATCH = 16384
_NW = 32
_BPW = _BATCH // _NW
_CHUNK = 128
_NCHUNK = _BPW // _CHUNK
_GPC = _CHUNK // 16
_MARGIN = 1.0


_NBLK_FULL = _NUM_CLASSES // 128
_TAIL = _NUM_CLASSES - _NBLK_FULL * 128
_PROWS = _NUM_CLASSES // 2


def _transpose_body(cT_hbm, pairs_hbm,
                    blk0, blk1, outc0, outc1, blkT, outcT,
                    si0, si1, so0, so1):
    wid = lax.axis_index("s") * 2 + lax.axis_index("c")
    nblk = jnp.where(wid <= 12, 25, 24)
    blks = (blk0, blk1)
    outs = (outc0, outc1)
    sis = (si0, si1)
    sos = (so0, so1)
    lane = lax.iota(jnp.int32, 16)
    rowq = [jnp.bitwise_and(q * 16 + lane, 63) for q in range(8)]
    colq = [lax.shift_right_logical(q * 16 + lane, 6) for q in range(8)]

    for p in range(2):
        pltpu.async_copy(cT_hbm.at[:, pl.ds((wid + 32 * p) * 128, 128)],
                         blks[p], sis[p])

    def super_step(ts, carry):
        for p in range(2):
            t = ts * 2 + p

            @pl.when(t < nblk)
            def _():
                b = wid + 32 * t
                pltpu.make_async_copy(
                    cT_hbm.at[:, pl.ds(0, 128)], blks[p], sis[p]).wait()

                @pl.when(t >= 2)
                def _():
                    pltpu.make_async_copy(
                        outs[p], pairs_hbm.at[pl.ds(0, 64)], sos[p]).wait()

                def trow(pr, carry):
                    p2 = pr * 2
                    for q in range(8):
                        v = plsc.load_gather(blks[p], [rowq[q], colq[q] + p2])
                        outs[p][pr, pl.ds(q * 16, 16)] = v
                    return carry

                lax.fori_loop(0, _FEAT_DIM, trow, 0)
                pltpu.async_copy(outs[p], pairs_hbm.at[pl.ds(b * 64, 64)],
                                 sos[p])

                @pl.when(t + 2 < nblk)
                def _():
                    pltpu.async_copy(
                        cT_hbm.at[:, pl.ds((b + 64) * 128, 128)],
                        blks[p], sis[p])
        return carry

    trip = lax.div(nblk + 1, 2)
    lax.fori_loop(0, trip, super_step, 0)
    for p in range(2):
        pltpu.make_async_copy(
            outs[p], pairs_hbm.at[pl.ds(0, 64)], sos[p]).wait()

    @pl.when(wid == 31)
    def _():
        pltpu.sync_copy(cT_hbm.at[:, pl.ds(_NBLK_FULL * 128, _TAIL)], blkT)
        for pr in range(_TAIL // 2):
            for q in range(8):
                v = plsc.load_gather(blkT, [rowq[q], colq[q] + pr * 2])
                outcT[pr, pl.ds(q * 16, 16)] = v
        pltpu.sync_copy(outcT, pairs_hbm.at[pl.ds(_PROWS - _TAIL // 2,
                                                  _TAIL // 2)])


_sc_transpose = pl.kernel(
    _transpose_body,
    mesh=plsc.VectorSubcoreMesh(core_axis_name="c", subcore_axis_name="s"),
    compiler_params=pltpu.CompilerParams(needs_layout_passes=False),
    out_type=jax.ShapeDtypeStruct((_PROWS, 2 * _FEAT_DIM), jnp.float32),
    scratch_types=[
        pltpu.VMEM((_FEAT_DIM, 128), jnp.float32),
        pltpu.VMEM((_FEAT_DIM, 128), jnp.float32),
        pltpu.VMEM((_FEAT_DIM, 2 * _FEAT_DIM), jnp.float32),
        pltpu.VMEM((_FEAT_DIM, 2 * _FEAT_DIM), jnp.float32),
        pltpu.VMEM((_FEAT_DIM, _TAIL), jnp.float32),
        pltpu.VMEM((_TAIL // 2, 2 * _FEAT_DIM), jnp.float32),
        pltpu.SemaphoreType.DMA,
        pltpu.SemaphoreType.DMA,
        pltpu.SemaphoreType.DMA,
        pltpu.SemaphoreType.DMA,
    ],
)


def _loss_body(label_hbm, featT_hbm, pairs_hbm, out_hbm,
               lbl_v, idx2_v, rows_v, featT_v, acc_v, sem):
    wid = lax.axis_index("s") * 2 + lax.axis_index("c")
    base = wid * _BPW

    pltpu.sync_copy(label_hbm.at[pl.ds(base, _BPW)], lbl_v)
    for t in range(_BPW // 16):
        idx2_v[pl.ds(t * 16, 16)] = lax.shift_right_logical(
            lbl_v[pl.ds(t * 16, 16)], 1)
    copies = [
        pltpu.async_copy(pairs_hbm.at[idx2_v.at[pl.ds(j * _CHUNK, _CHUNK)]],
                         rows_v.at[pl.ds(j * _CHUNK, _CHUNK)], sem)
        for j in range(_NCHUNK)
    ]
    pltpu.sync_copy(featT_hbm.at[:, pl.ds(base, _BPW)], featT_v)

    lane = lax.iota(jnp.int32, 16)
    acc0 = jnp.zeros((16,), jnp.float32)

    def make_group(j):
        def group(gi, acc):
            g16 = j * _CHUNK + gi * 16
            rows16 = g16 + lane
            lbl16 = lbl_v[pl.ds(g16, 16)]
            par64 = lax.shift_left(jnp.bitwise_and(lbl16, 1), 6)
            s = jnp.zeros((16,), jnp.float32)
            ff = jnp.zeros((16,), jnp.float32)
            dot = jnp.zeros((16,), jnp.float32)
            for k in range(_FEAT_DIM):
                c = plsc.load_gather(rows_v, [rows16, par64 + k])
                f = featT_v[k, pl.ds(g16, 16)]
                s = s + c * c
                ff = ff + f * f
                dot = dot + f * c
            sc = jnp.maximum(s, jnp.float32(1e-24))
            seed = jnp.int32(0x5F3759DF) - lax.shift_right_arithmetic(
                lax.bitcast_convert_type(sc, jnp.int32), 1)
            y = lax.bitcast_convert_type(seed, jnp.float32)
            for _ in range(3):
                y = y * (jnp.float32(1.5) - jnp.float32(0.5) * sc * y * y)
            d = ff - 2.0 * (dot * y) + s * (y * y) - _MARGIN
            return acc + jnp.maximum(jnp.exp(d) - 1.0, 0.0)
        return group

    acc = acc0
    for j in range(_NCHUNK):
        copies[j].wait()
        acc = lax.fori_loop(0, _GPC, make_group(j), acc)

    acc_v[...] = acc
    pltpu.sync_copy(acc_v, out_hbm.at[pl.ds(wid * 16, 16)])


_sc_loss = pl.kernel(
    _loss_body,
    mesh=plsc.VectorSubcoreMesh(core_axis_name="c", subcore_axis_name="s"),
    compiler_params=pltpu.CompilerParams(needs_layout_passes=False),
    out_type=jax.ShapeDtypeStruct((_NW * 16,), jnp.float32),
    scratch_types=[
        pltpu.VMEM((_BPW,), jnp.int32),
        pltpu.VMEM((_BPW,), jnp.int32),
        pltpu.VMEM((_BPW, 2 * _FEAT_DIM), jnp.float32),
        pltpu.VMEM((_FEAT_DIM, _BPW), jnp.float32),
        pltpu.VMEM((16,), jnp.float32),
        pltpu.SemaphoreType.DMA,
    ],
)


def kernel(label, feat, centers):
    pairs = _sc_transpose(centers.T)
    partials = _sc_loss(label.astype(jnp.int32), feat.T, pairs)
    return jnp.sum(partials) / 2.0 / _BATCH


# speedup vs baseline: 1.0016x; 1.0016x over previous
import jax
import jax.numpy as jnp
from jax import lax
from jax.experimental import pallas as pl
from jax.experimental.pallas import tpu as pltpu
from jax.experimental.pallas import tpu_sc as plsc

_NUM_CLASSES = 100000
_FEAT_DIM = 64
_BATCH = 16384
_NW = 32
_BPW = _BATCH // _NW
_CHUNK = 128
_NCHUNK = _BPW // _CHUNK
_GPC = _CHUNK // 16
_MARGIN = 1.0


_NBLK_FULL = _NUM_CLASSES // 128
_TAIL = _NUM_CLASSES - _NBLK_FULL * 128
_PROWS = _NUM_CLASSES // 2


def _transpose_body(cT_hbm, pairs_hbm,
                    blk0, blk1, outc0, outc1, blkT, outcT,
                    si0, si1, so0, so1):
    wid = lax.axis_index("s") * 2 + lax.axis_index("c")
    nblk = jnp.where(wid <= 12, 25, 24)
    blks = (blk0, blk1)
    outs = (outc0, outc1)
    sis = (si0, si1)
    sos = (so0, so1)
    lane = lax.iota(jnp.int32, 16)
    rowq = [jnp.bitwise_and(q * 16 + lane, 63) for q in range(8)]
    colq = [lax.shift_right_logical(q * 16 + lane, 6) for q in range(8)]

    def fire_block(b, p):
        for tr in range(8):
            pltpu.async_copy(
                cT_hbm.at[pl.ds(tr * 8, 8), pl.ds(b * 128, 128)],
                blks[p].at[pl.ds(tr * 8, 8), :], sis[p])

    for p in range(2):
        fire_block(wid + 32 * p, p)

    def super_step(ts, carry):
        for p in range(2):
            t = ts * 2 + p

            @pl.when(t < nblk)
            def _():
                b = wid + 32 * t
                pltpu.make_async_copy(
                    cT_hbm.at[:, pl.ds(0, 128)], blks[p], sis[p]).wait()

                @pl.when(t >= 2)
                def _():
                    pltpu.make_async_copy(
                        outs[p], pairs_hbm.at[pl.ds(0, 64)], sos[p]).wait()

                def trow(pr, carry):
                    p2 = pr * 2
                    for q in range(8):
                        v = plsc.load_gather(blks[p], [rowq[q], colq[q] + p2])
                        outs[p][pr, pl.ds(q * 16, 16)] = v
                    return carry

                lax.fori_loop(0, _FEAT_DIM, trow, 0)
                pltpu.async_copy(outs[p], pairs_hbm.at[pl.ds(b * 64, 64)],
                                 sos[p])

                @pl.when(t + 2 < nblk)
                def _():
                    fire_block(b + 64, p)
        return carry

    trip = lax.div(nblk + 1, 2)
    lax.fori_loop(0, trip, super_step, 0)
    for p in range(2):
        pltpu.make_async_copy(
            outs[p], pairs_hbm.at[pl.ds(0, 64)], sos[p]).wait()

    @pl.when(wid == 31)
    def _():
        pltpu.sync_copy(cT_hbm.at[:, pl.ds(_NBLK_FULL * 128, _TAIL)], blkT)
        for pr in range(_TAIL // 2):
            for q in range(8):
                v = plsc.load_gather(blkT, [rowq[q], colq[q] + pr * 2])
                outcT[pr, pl.ds(q * 16, 16)] = v
        pltpu.sync_copy(outcT, pairs_hbm.at[pl.ds(_PROWS - _TAIL // 2,
                                                  _TAIL // 2)])


_sc_transpose = pl.kernel(
    _transpose_body,
    mesh=plsc.VectorSubcoreMesh(core_axis_name="c", subcore_axis_name="s"),
    compiler_params=pltpu.CompilerParams(needs_layout_passes=False),
    out_type=jax.ShapeDtypeStruct((_PROWS, 2 * _FEAT_DIM), jnp.float32),
    scratch_types=[
        pltpu.VMEM((_FEAT_DIM, 128), jnp.float32),
        pltpu.VMEM((_FEAT_DIM, 128), jnp.float32),
        pltpu.VMEM((_FEAT_DIM, 2 * _FEAT_DIM), jnp.float32),
        pltpu.VMEM((_FEAT_DIM, 2 * _FEAT_DIM), jnp.float32),
        pltpu.VMEM((_FEAT_DIM, _TAIL), jnp.float32),
        pltpu.VMEM((_TAIL // 2, 2 * _FEAT_DIM), jnp.float32),
        pltpu.SemaphoreType.DMA,
        pltpu.SemaphoreType.DMA,
        pltpu.SemaphoreType.DMA,
        pltpu.SemaphoreType.DMA,
    ],
)


def _loss_body(label_hbm, featT_hbm, pairs_hbm, out_hbm,
               lbl_v, idx2_v, rows_v, featT_v, acc_v, sem):
    wid = lax.axis_index("s") * 2 + lax.axis_index("c")
    base = wid * _BPW

    pltpu.sync_copy(label_hbm.at[pl.ds(base, _BPW)], lbl_v)
    for t in range(_BPW // 16):
        idx2_v[pl.ds(t * 16, 16)] = lax.shift_right_logical(
            lbl_v[pl.ds(t * 16, 16)], 1)
    copies = [
        pltpu.async_copy(pairs_hbm.at[idx2_v.at[pl.ds(j * _CHUNK, _CHUNK)]],
                         rows_v.at[pl.ds(j * _CHUNK, _CHUNK)], sem)
        for j in range(_NCHUNK)
    ]
    pltpu.sync_copy(featT_hbm.at[:, pl.ds(base, _BPW)], featT_v)

    lane = lax.iota(jnp.int32, 16)
    acc0 = jnp.zeros((16,), jnp.float32)

    def make_group(j):
        def group(gi, acc):
            g16 = j * _CHUNK + gi * 16
            rows16 = g16 + lane
            lbl16 = lbl_v[pl.ds(g16, 16)]
            par64 = lax.shift_left(jnp.bitwise_and(lbl16, 1), 6)
            s = jnp.zeros((16,), jnp.float32)
            ff = jnp.zeros((16,), jnp.float32)
            dot = jnp.zeros((16,), jnp.float32)
            for k in range(_FEAT_DIM):
                c = plsc.load_gather(rows_v, [rows16, par64 + k])
                f = featT_v[k, pl.ds(g16, 16)]
                s = s + c * c
                ff = ff + f * f
                dot = dot + f * c
            sc = jnp.maximum(s, jnp.float32(1e-24))
            seed = jnp.int32(0x5F3759DF) - lax.shift_right_arithmetic(
                lax.bitcast_convert_type(sc, jnp.int32), 1)
            y = lax.bitcast_convert_type(seed, jnp.float32)
            for _ in range(3):
                y = y * (jnp.float32(1.5) - jnp.float32(0.5) * sc * y * y)
            d = ff - 2.0 * (dot * y) + s * (y * y) - _MARGIN
            return acc + jnp.maximum(jnp.exp(d) - 1.0, 0.0)
        return group

    acc = acc0
    for j in range(_NCHUNK):
        copies[j].wait()
        acc = lax.fori_loop(0, _GPC, make_group(j), acc)

    acc_v[...] = acc
    pltpu.sync_copy(acc_v, out_hbm.at[pl.ds(wid * 16, 16)])


_sc_loss = pl.kernel(
    _loss_body,
    mesh=plsc.VectorSubcoreMesh(core_axis_name="c", subcore_axis_name="s"),
    compiler_params=pltpu.CompilerParams(needs_layout_passes=False),
    out_type=jax.ShapeDtypeStruct((_NW * 16,), jnp.float32),
    scratch_types=[
        pltpu.VMEM((_BPW,), jnp.int32),
        pltpu.VMEM((_BPW,), jnp.int32),
        pltpu.VMEM((_BPW, 2 * _FEAT_DIM), jnp.float32),
        pltpu.VMEM((_FEAT_DIM, _BPW), jnp.float32),
        pltpu.VMEM((16,), jnp.float32),
        pltpu.SemaphoreType.DMA,
    ],
)


def kernel(label, feat, centers):
    pairs = _sc_transpose(centers.T)
    partials = _sc_loss(label.astype(jnp.int32), feat.T, pairs)
    return jnp.sum(partials) / 2.0 / _BATCH


# speedup vs baseline: 1.9286x; 1.9255x over previous
import jax
import jax.numpy as jnp
from jax import lax
from jax.experimental import pallas as pl
from jax.experimental.pallas import tpu as pltpu
from jax.experimental.pallas import tpu_sc as plsc

_NUM_CLASSES = 100000
_FEAT_DIM = 64
_BATCH = 16384
_NW = 32
_BPW = _BATCH // _NW
_CHUNK = 128
_NCHUNK = _BPW // _CHUNK
_GPC = _CHUNK // 16
_MARGIN = 1.0


_NBLK_FULL = _NUM_CLASSES // 128
_TAIL = _NUM_CLASSES - _NBLK_FULL * 128
_PROWS = _NUM_CLASSES // 2


def _transpose_body(cT_hbm, pairs_hbm,
                    blk0, blk1, outc0, outc1, blkT, outcT,
                    si0, si1, so0, so1):
    wid = lax.axis_index("s") * 2 + lax.axis_index("c")
    nblk = jnp.where(wid <= 12, 25, 24)
    blks = (blk0, blk1)
    outs = (outc0, outc1)
    sis = (si0, si1)
    sos = (so0, so1)
    lane = lax.iota(jnp.int32, 16)
    rowq = [jnp.bitwise_and(q * 16 + lane, 63) for q in range(8)]
    colq = [lax.shift_right_logical(q * 16 + lane, 6) for q in range(8)]

    def fire_block(b, p):
        for tr in range(8):
            pltpu.async_copy(
                cT_hbm.at[pl.ds(tr * 8, 8), pl.ds(b * 128, 128)],
                blks[p].at[pl.ds(tr * 8, 8), :], sis[p])

    for p in range(2):
        fire_block(wid + 32 * p, p)

    def super_step(ts, carry):
        for p in range(2):
            t = ts * 2 + p

            @pl.when(t < nblk)
            def _():
                b = wid + 32 * t
                pltpu.make_async_copy(
                    cT_hbm.at[:, pl.ds(0, 128)], blks[p], sis[p]).wait()

                @pl.when(t >= 2)
                def _():
                    pltpu.make_async_copy(
                        outs[p], pairs_hbm.at[pl.ds(0, 64)], sos[p]).wait()

                def trow(pr, carry):
                    p2 = pr * 2
                    for q in range(8):
                        v = plsc.load_gather(blks[p], [rowq[q], colq[q] + p2])
                        outs[p][pr, pl.ds(q * 16, 16)] = v
                    return carry

                lax.fori_loop(0, _FEAT_DIM, trow, 0)
                pltpu.async_copy(outs[p], pairs_hbm.at[pl.ds(b * 64, 64)],
                                 sos[p])

                @pl.when(t + 2 < nblk)
                def _():
                    fire_block(b + 64, p)
        return carry

    trip = lax.div(nblk + 1, 2)
    lax.fori_loop(0, trip, super_step, 0)
    for p in range(2):
        pltpu.make_async_copy(
            outs[p], pairs_hbm.at[pl.ds(0, 64)], sos[p]).wait()

    @pl.when(wid == 31)
    def _():
        pltpu.sync_copy(cT_hbm.at[:, pl.ds(_NBLK_FULL * 128, _TAIL)], blkT)
        for pr in range(_TAIL // 2):
            for q in range(8):
                v = plsc.load_gather(blkT, [rowq[q], colq[q] + pr * 2])
                outcT[pr, pl.ds(q * 16, 16)] = v
        pltpu.sync_copy(outcT, pairs_hbm.at[pl.ds(_PROWS - _TAIL // 2,
                                                  _TAIL // 2)])


_sc_transpose = pl.kernel(
    _transpose_body,
    mesh=plsc.VectorSubcoreMesh(core_axis_name="c", subcore_axis_name="s"),
    compiler_params=pltpu.CompilerParams(needs_layout_passes=False),
    out_type=jax.ShapeDtypeStruct((_PROWS, 2 * _FEAT_DIM), jnp.float32),
    scratch_types=[
        pltpu.VMEM((_FEAT_DIM, 128), jnp.float32),
        pltpu.VMEM((_FEAT_DIM, 128), jnp.float32),
        pltpu.VMEM((_FEAT_DIM, 2 * _FEAT_DIM), jnp.float32),
        pltpu.VMEM((_FEAT_DIM, 2 * _FEAT_DIM), jnp.float32),
        pltpu.VMEM((_FEAT_DIM, _TAIL), jnp.float32),
        pltpu.VMEM((_TAIL // 2, 2 * _FEAT_DIM), jnp.float32),
        pltpu.SemaphoreType.DMA,
        pltpu.SemaphoreType.DMA,
        pltpu.SemaphoreType.DMA,
        pltpu.SemaphoreType.DMA,
    ],
)


def _loss_body(label_hbm, featT_hbm, pairs_hbm, out_hbm,
               lbl_v, idx2_v, rows_v, featT_v, acc_v, sem):
    wid = lax.axis_index("s") * 2 + lax.axis_index("c")
    base = wid * _BPW

    pltpu.sync_copy(label_hbm.at[pl.ds(base, _BPW)], lbl_v)
    for t in range(_BPW // 16):
        idx2_v[pl.ds(t * 16, 16)] = lax.shift_right_logical(
            lbl_v[pl.ds(t * 16, 16)], 1)
    copies = [
        pltpu.async_copy(pairs_hbm.at[idx2_v.at[pl.ds(j * _CHUNK, _CHUNK)]],
                         rows_v.at[pl.ds(j * _CHUNK, _CHUNK)], sem)
        for j in range(_NCHUNK)
    ]
    pltpu.sync_copy(featT_hbm.at[:, pl.ds(base, _BPW)], featT_v)

    lane = lax.iota(jnp.int32, 16)
    acc0 = jnp.zeros((16,), jnp.float32)

    def make_group(j):
        def group(gi, acc):
            g16 = j * _CHUNK + gi * 16
            rows16 = g16 + lane
            lbl16 = lbl_v[pl.ds(g16, 16)]
            par64 = lax.shift_left(jnp.bitwise_and(lbl16, 1), 6)
            s = jnp.zeros((16,), jnp.float32)
            ff = jnp.zeros((16,), jnp.float32)
            dot = jnp.zeros((16,), jnp.float32)
            for k in range(_FEAT_DIM):
                c = plsc.load_gather(rows_v, [rows16, par64 + k])
                f = featT_v[k, pl.ds(g16, 16)]
                s = s + c * c
                ff = ff + f * f
                dot = dot + f * c
            sc = jnp.maximum(s, jnp.float32(1e-24))
            seed = jnp.int32(0x5F3759DF) - lax.shift_right_arithmetic(
                lax.bitcast_convert_type(sc, jnp.int32), 1)
            y = lax.bitcast_convert_type(seed, jnp.float32)
            for _ in range(3):
                y = y * (jnp.float32(1.5) - jnp.float32(0.5) * sc * y * y)
            d = ff - 2.0 * (dot * y) + s * (y * y) - _MARGIN
            return acc + jnp.maximum(jnp.exp(d) - 1.0, 0.0)
        return group

    acc = acc0
    for j in range(_NCHUNK):
        copies[j].wait()
        acc = lax.fori_loop(0, _GPC, make_group(j), acc)

    acc_v[...] = acc
    pltpu.sync_copy(acc_v, out_hbm.at[pl.ds(wid * 16, 16)])


_sc_loss = pl.kernel(
    _loss_body,
    mesh=plsc.VectorSubcoreMesh(core_axis_name="c", subcore_axis_name="s"),
    compiler_params=pltpu.CompilerParams(needs_layout_passes=False),
    out_type=jax.ShapeDtypeStruct((_NW * 16,), jnp.float32),
    scratch_types=[
        pltpu.VMEM((_BPW,), jnp.int32),
        pltpu.VMEM((_BPW,), jnp.int32),
        pltpu.VMEM((_BPW, 2 * _FEAT_DIM), jnp.float32),
        pltpu.VMEM((_FEAT_DIM, _BPW), jnp.float32),
        pltpu.VMEM((16,), jnp.float32),
        pltpu.SemaphoreType.DMA,
    ],
)


def kernel(label, feat, centers):
    pairs = centers.reshape(_NUM_CLASSES // 2, 2 * _FEAT_DIM)
    partials = _sc_loss(label.astype(jnp.int32), feat.T, pairs)
    return jnp.sum(partials) / 2.0 / _BATCH


# speedup vs baseline: 2.1316x; 1.1053x over previous
import jax
import jax.numpy as jnp
from jax import lax
from jax.experimental import pallas as pl
from jax.experimental.pallas import tpu as pltpu
from jax.experimental.pallas import tpu_sc as plsc

_NUM_CLASSES = 100000
_FEAT_DIM = 64
_BATCH = 16384
_NW = 32
_BPW = _BATCH // _NW
_CHUNK = 128
_NCHUNK = _BPW // _CHUNK
_GPC = _CHUNK // 16
_MARGIN = 1.0


_NBLK_FULL = _NUM_CLASSES // 128
_TAIL = _NUM_CLASSES - _NBLK_FULL * 128
_PROWS = _NUM_CLASSES // 2


def _transpose_body(cT_hbm, pairs_hbm,
                    blk0, blk1, outc0, outc1, blkT, outcT,
                    si0, si1, so0, so1):
    wid = lax.axis_index("s") * 2 + lax.axis_index("c")
    nblk = jnp.where(wid <= 12, 25, 24)
    blks = (blk0, blk1)
    outs = (outc0, outc1)
    sis = (si0, si1)
    sos = (so0, so1)
    lane = lax.iota(jnp.int32, 16)
    rowq = [jnp.bitwise_and(q * 16 + lane, 63) for q in range(8)]
    colq = [lax.shift_right_logical(q * 16 + lane, 6) for q in range(8)]

    def fire_block(b, p):
        for tr in range(8):
            pltpu.async_copy(
                cT_hbm.at[pl.ds(tr * 8, 8), pl.ds(b * 128, 128)],
                blks[p].at[pl.ds(tr * 8, 8), :], sis[p])

    for p in range(2):
        fire_block(wid + 32 * p, p)

    def super_step(ts, carry):
        for p in range(2):
            t = ts * 2 + p

            @pl.when(t < nblk)
            def _():
                b = wid + 32 * t
                pltpu.make_async_copy(
                    cT_hbm.at[:, pl.ds(0, 128)], blks[p], sis[p]).wait()

                @pl.when(t >= 2)
                def _():
                    pltpu.make_async_copy(
                        outs[p], pairs_hbm.at[pl.ds(0, 64)], sos[p]).wait()

                def trow(pr, carry):
                    p2 = pr * 2
                    for q in range(8):
                        v = plsc.load_gather(blks[p], [rowq[q], colq[q] + p2])
                        outs[p][pr, pl.ds(q * 16, 16)] = v
                    return carry

                lax.fori_loop(0, _FEAT_DIM, trow, 0)
                pltpu.async_copy(outs[p], pairs_hbm.at[pl.ds(b * 64, 64)],
                                 sos[p])

                @pl.when(t + 2 < nblk)
                def _():
                    fire_block(b + 64, p)
        return carry

    trip = lax.div(nblk + 1, 2)
    lax.fori_loop(0, trip, super_step, 0)
    for p in range(2):
        pltpu.make_async_copy(
            outs[p], pairs_hbm.at[pl.ds(0, 64)], sos[p]).wait()

    @pl.when(wid == 31)
    def _():
        pltpu.sync_copy(cT_hbm.at[:, pl.ds(_NBLK_FULL * 128, _TAIL)], blkT)
        for pr in range(_TAIL // 2):
            for q in range(8):
                v = plsc.load_gather(blkT, [rowq[q], colq[q] + pr * 2])
                outcT[pr, pl.ds(q * 16, 16)] = v
        pltpu.sync_copy(outcT, pairs_hbm.at[pl.ds(_PROWS - _TAIL // 2,
                                                  _TAIL // 2)])


_sc_transpose = pl.kernel(
    _transpose_body,
    mesh=plsc.VectorSubcoreMesh(core_axis_name="c", subcore_axis_name="s"),
    compiler_params=pltpu.CompilerParams(needs_layout_passes=False),
    out_type=jax.ShapeDtypeStruct((_PROWS, 2 * _FEAT_DIM), jnp.float32),
    scratch_types=[
        pltpu.VMEM((_FEAT_DIM, 128), jnp.float32),
        pltpu.VMEM((_FEAT_DIM, 128), jnp.float32),
        pltpu.VMEM((_FEAT_DIM, 2 * _FEAT_DIM), jnp.float32),
        pltpu.VMEM((_FEAT_DIM, 2 * _FEAT_DIM), jnp.float32),
        pltpu.VMEM((_FEAT_DIM, _TAIL), jnp.float32),
        pltpu.VMEM((_TAIL // 2, 2 * _FEAT_DIM), jnp.float32),
        pltpu.SemaphoreType.DMA,
        pltpu.SemaphoreType.DMA,
        pltpu.SemaphoreType.DMA,
        pltpu.SemaphoreType.DMA,
    ],
)


def _loss_body(label_hbm, featT_hbm, pairs_hbm, out_hbm,
               lbl_v, rows_v, featT_v, acc_v, sem):
    wid = lax.axis_index("s") * 2 + lax.axis_index("c")
    base = wid * _BPW

    pltpu.sync_copy(label_hbm.at[pl.ds(base, _BPW)], lbl_v)
    copies = [
        pltpu.async_copy(pairs_hbm.at[lbl_v.at[pl.ds(j * _CHUNK, _CHUNK)]],
                         rows_v.at[pl.ds(j * _CHUNK, _CHUNK)], sem)
        for j in range(_NCHUNK)
    ]
    pltpu.sync_copy(featT_hbm.at[:, pl.ds(base, _BPW)], featT_v)

    lane = lax.iota(jnp.int32, 16)
    acc0 = jnp.zeros((16,), jnp.float32)

    def make_group(j):
        def group(gi, acc):
            g16 = j * _CHUNK + gi * 16
            rows16 = g16 + lane

            s = jnp.zeros((16,), jnp.float32)
            ff = jnp.zeros((16,), jnp.float32)
            dot = jnp.zeros((16,), jnp.float32)
            for k in range(_FEAT_DIM):
                c = plsc.load_gather(rows_v, [rows16, jnp.full((16,), k, jnp.int32)])
                f = featT_v[k, pl.ds(g16, 16)]
                s = s + c * c
                ff = ff + f * f
                dot = dot + f * c
            sc = jnp.maximum(s, jnp.float32(1e-24))
            seed = jnp.int32(0x5F3759DF) - lax.shift_right_arithmetic(
                lax.bitcast_convert_type(sc, jnp.int32), 1)
            y = lax.bitcast_convert_type(seed, jnp.float32)
            for _ in range(3):
                y = y * (jnp.float32(1.5) - jnp.float32(0.5) * sc * y * y)
            d = ff - 2.0 * (dot * y) + s * (y * y) - _MARGIN
            return acc + jnp.maximum(jnp.exp(d) - 1.0, 0.0)
        return group

    acc = acc0
    for j in range(_NCHUNK):
        copies[j].wait()
        acc = lax.fori_loop(0, _GPC, make_group(j), acc)

    acc_v[...] = acc
    pltpu.sync_copy(acc_v, out_hbm.at[pl.ds(wid * 16, 16)])


_sc_loss = pl.kernel(
    _loss_body,
    mesh=plsc.VectorSubcoreMesh(core_axis_name="c", subcore_axis_name="s"),
    compiler_params=pltpu.CompilerParams(needs_layout_passes=False),
    out_type=jax.ShapeDtypeStruct((_NW * 16,), jnp.float32),
    scratch_types=[
        pltpu.VMEM((_BPW,), jnp.int32),
        pltpu.VMEM((_BPW, 2 * _FEAT_DIM), jnp.float32),
        pltpu.VMEM((_FEAT_DIM, _BPW), jnp.float32),
        pltpu.VMEM((16,), jnp.float32),
        pltpu.SemaphoreType.DMA,
    ],
)


def kernel(label, feat, centers):
    pairs = jnp.pad(centers, ((0, 0), (0, _FEAT_DIM)))
    partials = _sc_loss(label.astype(jnp.int32), feat.T, pairs)
    return jnp.sum(partials) / 2.0 / _BATCH
